# Initial kernel scaffold; baseline (speedup 1.0000x reference)
import jax
import jax.numpy as jnp
from jax.experimental import pallas as pl


def kernel(s_p, v_p, s_m, v_m, s_w, v_w, pos_p, pos_w, pos_m, params):
    raise NotImplementedError("write your pallas kernel here")


# speedup vs baseline: 10.0824x; 10.0824x over previous
import functools

import jax
import jax.numpy as jnp
from jax import lax
from jax.experimental import pallas as pl
from jax.experimental.pallas import tpu as pltpu
from jax.experimental.pallas import tpu_sc as plsc

S = 256
V = 32
RBF = 16
KPW = 12
KWW = 8
NWAT = 4096
F32 = jnp.float32
DMSG = S + 3 * V


def _full_spec(shape):
    nd = len(shape)
    return pl.BlockSpec(shape, lambda i: (0,) * nd)


def _knn_body(ps_ref, pwt_ref, out_ref, *, k, n_dst, tile, exclude_self):
    i = pl.program_id(0)
    ps = ps_ref[...]
    pwt = pwt_ref[...]
    sn = jnp.sum(ps * ps, axis=1, keepdims=True)
    wn = jnp.sum(pwt * pwt, axis=0, keepdims=True)
    d2 = sn + wn - 2.0 * jnp.dot(ps, pwt, preferred_element_type=F32)
    col = lax.broadcasted_iota(jnp.int32, d2.shape, 1)
    inf = jnp.float32(jnp.inf)
    if exclude_self:
        row = lax.broadcasted_iota(jnp.int32, d2.shape, 0) + i * tile
        d2 = jnp.where(col == row, inf, d2)
    cols = []
    for _ in range(k):
        m = jnp.min(d2, axis=1, keepdims=True)
        am = jnp.min(jnp.where(d2 <= m, col, n_dst), axis=1, keepdims=True)
        cols.append(am)
        d2 = jnp.where(col == am, inf, d2)
    out_ref[...] = jnp.concatenate(cols, axis=1)


def _knn(pos_src8, pos_dst_t, k, exclude_self):
    n_src = pos_src8.shape[0]
    n_dst = pos_dst_t.shape[1]
    tile = 128
    return pl.pallas_call(
        functools.partial(_knn_body, k=k, n_dst=n_dst, tile=tile,
                          exclude_self=exclude_self),
        grid=(n_src // tile,),
        in_specs=[pl.BlockSpec((tile, 8), lambda i: (i, 0)),
                  _full_spec((8, n_dst))],
        out_specs=pl.BlockSpec((tile, k), lambda i: (i, 0)),
        out_shape=jax.ShapeDtypeStruct((n_src, k), jnp.int32),
    )(pos_src8, pos_dst_t)


def _mm_body(x_ref, w_ref, o_ref):
    o_ref[...] = jnp.dot(x_ref[...], w_ref[...], preferred_element_type=F32)


def _mm(x, w, tile=256):
    n, din = x.shape
    dout = w.shape[1]
    return pl.pallas_call(
        _mm_body,
        grid=(n // tile,),
        in_specs=[pl.BlockSpec((tile, din), lambda i: (i, 0)),
                  _full_spec((din, dout))],
        out_specs=pl.BlockSpec((tile, dout), lambda i: (i, 0)),
        out_shape=jax.ShapeDtypeStruct((n, dout), F32),
    )(x, w)


def _rep_mat(e_t, s_tile, k):
    er = lax.broadcasted_iota(jnp.int32, (e_t, s_tile), 0) // k
    sc = lax.broadcasted_iota(jnp.int32, (e_t, s_tile), 1)
    return (er == sc).astype(F32)


def _es_body(ps_ref, pd_ref, o_ref, *, k, s_tile):
    e_t = s_tile * k
    rep = _rep_mat(e_t, s_tile, k)
    psrc = jnp.dot(rep, ps_ref[...], preferred_element_type=F32)
    dvec = pd_ref[:, 0:3] - psrc[:, 0:3]
    dist = jnp.sqrt(jnp.sum(dvec * dvec, axis=1, keepdims=True) + 1e-8)
    unit = dvec / dist
    mu = lax.broadcasted_iota(jnp.int32, (1, RBF), 1).astype(F32) * (
        20.0 / (RBF - 1))
    sig = 20.0 / RBF
    rb = jnp.exp(-((dist - mu) ** 2) / (2.0 * sig * sig))
    pad = jnp.zeros((e_t, 13), F32)
    o_ref[...] = jnp.concatenate([unit, rb, pad], axis=1)


def _edge_static(pos_src8, pos_dst_g, k, s_tile):
    n_src = pos_src8.shape[0]
    e_t = s_tile * k
    return pl.pallas_call(
        functools.partial(_es_body, k=k, s_tile=s_tile),
        grid=(n_src // s_tile,),
        in_specs=[pl.BlockSpec((s_tile, 8), lambda i: (i, 0)),
                  pl.BlockSpec((e_t, 16), lambda i: (i, 0))],
        out_specs=pl.BlockSpec((e_t, 32), lambda i: (i, 0)),
        out_shape=jax.ShapeDtypeStruct((n_src * k, 32), F32),
    )(pos_src8, pos_dst_g)


def _msg_body(bsrc_ref, vsrc_ref, g_ref, es_ref,
              w1_ref, wh1_ref, wv1_ref, wsv1_ref,
              ws2_ref, wh2_ref, wv2_ref, wsv2_ref,
              ws3_ref, wh3_ref, wv3_ref, wsv3_ref,
              bias_ref, o_ref, *, k, s_tile):
    e_t = s_tile * k
    rep = _rep_mat(e_t, s_tile, k)
    bsrc = jnp.dot(rep, bsrc_ref[...], preferred_element_type=F32)
    vsrc = jnp.dot(rep, vsrc_ref[...], preferred_element_type=F32)
    g = g_ref[...]
    bdst = g[:, :S]
    vdst = g[:, S:S + 3 * V]
    es = es_ref[...]
    unit = es[:, 0:3]
    rb = es[:, 3:3 + RBF]
    bias = bias_ref[...]
    bs1, bs2, bs3 = bias[0:1, :], bias[1:2, :], bias[2:3, :]
    bsv1, bsv2, bsv3 = bias[3:4, 0:V], bias[4:5, 0:V], bias[5:6, 0:V]

    wh1 = wh1_ref[...]
    vh = []
    for c in range(3):
        vcat = jnp.concatenate(
            [vsrc[:, c * V:(c + 1) * V], unit[:, c:c + 1],
             vdst[:, c * V:(c + 1) * V]], axis=1)
        vh.append(jnp.dot(vcat, wh1, preferred_element_type=F32))
    vn1 = jnp.sqrt(vh[0] * vh[0] + vh[1] * vh[1] + vh[2] * vh[2] + 1e-8)
    s1p = bsrc + bdst + bs1 + jnp.dot(
        jnp.concatenate([rb, vn1], axis=1), w1_ref[...],
        preferred_element_type=F32)
    gate1 = jax.nn.sigmoid(
        jnp.dot(jax.nn.sigmoid(s1p), wsv1_ref[...],
                preferred_element_type=F32) + bsv1)
    wv1 = wv1_ref[...]
    v1 = [jnp.dot(vh[c], wv1, preferred_element_type=F32) * gate1
          for c in range(3)]
    s1 = jnp.maximum(s1p, 0.0)

    wh2 = wh2_ref[...]
    vh2 = [jnp.dot(v1[c], wh2, preferred_element_type=F32) for c in range(3)]
    vn2 = jnp.sqrt(vh2[0] * vh2[0] + vh2[1] * vh2[1] + vh2[2] * vh2[2] + 1e-8)
    s2p = jnp.dot(jnp.concatenate([s1, vn2], axis=1), ws2_ref[...],
                  preferred_element_type=F32) + bs2
    gate2 = jax.nn.sigmoid(
        jnp.dot(jax.nn.sigmoid(s2p), wsv2_ref[...],
                preferred_element_type=F32) + bsv2)
    wv2 = wv2_ref[...]
    v2 = [jnp.dot(vh2[c], wv2, preferred_element_type=F32) * gate2
          for c in range(3)]
    s2 = jnp.maximum(s2p, 0.0)

    wh3 = wh3_ref[...]
    vh3 = [jnp.dot(v2[c], wh3, preferred_element_type=F32) for c in range(3)]
    vn3 = jnp.sqrt(vh3[0] * vh3[0] + vh3[1] * vh3[1] + vh3[2] * vh3[2] + 1e-8)
    s3 = jnp.dot(jnp.concatenate([s2, vn3], axis=1), ws3_ref[...],
                 preferred_element_type=F32) + bs3
    gate3 = jax.nn.sigmoid(
        jnp.dot(s3, wsv3_ref[...], preferred_element_type=F32) + bsv3)
    wv3 = wv3_ref[...]
    v3 = [jnp.dot(vh3[c], wv3, preferred_element_type=F32) * gate3
          for c in range(3)]
    o_ref[...] = jnp.concatenate([s3, v3[0], v3[1], v3[2]], axis=1)


def _msg(wargs, bsrc, vsrc_cm, gth, es, k, s_tile):
    n_src = bsrc.shape[0]
    e_t = s_tile * k
    in_specs = [
        pl.BlockSpec((s_tile, S), lambda i: (i, 0)),
        pl.BlockSpec((s_tile, 3 * V), lambda i: (i, 0)),
        pl.BlockSpec((e_t, DMSG), lambda i: (i, 0)),
        pl.BlockSpec((e_t, 32), lambda i: (i, 0)),
    ] + [_full_spec(w.shape) for w in wargs]
    return pl.pallas_call(
        functools.partial(_msg_body, k=k, s_tile=s_tile),
        grid=(n_src // s_tile,),
        in_specs=in_specs,
        out_specs=pl.BlockSpec((e_t, DMSG), lambda i: (i, 0)),
        out_shape=jax.ShapeDtypeStruct((n_src * k, DMSG), F32),
    )(bsrc, vsrc_cm, gth, es, *wargs)


def _prep_msg_weights(chain):
    p1, p2, p3 = chain
    ws = p1["ws"]
    w_src = ws[0:S]
    w_rbf = ws[S:S + RBF]
    w_dst = ws[S + RBF:2 * S + RBF]
    w_vn = ws[2 * S + RBF:]
    w1 = jnp.concatenate([w_rbf, w_vn], axis=0)

    def padv(b):
        return jnp.concatenate([b, jnp.zeros((S - V,), F32)])

    bias = jnp.stack([p1["bs"], p2["bs"], p3["bs"],
                      padv(p1["bsv"]), padv(p2["bsv"]), padv(p3["bsv"])])
    wargs = (w1, p1["wh"], p1["wv"], p1["wsv"],
             p2["ws"], p2["wh"], p2["wv"], p2["wsv"],
             p3["ws"], p3["wh"], p3["wv"], p3["wsv"], bias)
    return w_src, w_dst, wargs


def _upd_body(s_ref, v_ref, p0_ref, p1_ref,
              ws1_ref, wh1_ref, wv1_ref, wsv1_ref,
              ws2_ref, wh2_ref, wv2_ref, wsv2_ref,
              ws3_ref, wh3_ref, wv3_ref, wsv3_ref,
              bias_ref, os_ref, ov_ref):
    s = s_ref[...]
    vcm = v_ref[...]
    agg = p0_ref[...] + p1_ref[...]
    aggs = agg[:, :S]
    aggv = agg[:, S:]
    bias = bias_ref[...]
    bs1, bs2, bs3 = bias[0:1, :], bias[1:2, :], bias[2:3, :]
    bsv1, bsv2, bsv3 = bias[3:4, 0:V], bias[4:5, 0:V], bias[5:6, 0:V]

    wh1 = wh1_ref[...]
    vh = []
    for c in range(3):
        uv = jnp.concatenate(
            [vcm[:, c * V:(c + 1) * V], aggv[:, c * V:(c + 1) * V]], axis=1)
        vh.append(jnp.dot(uv, wh1, preferred_element_type=F32))
    vn1 = jnp.sqrt(vh[0] * vh[0] + vh[1] * vh[1] + vh[2] * vh[2] + 1e-8)
    s1p = jnp.dot(jnp.concatenate([s, aggs, vn1], axis=1), ws1_ref[...],
                  preferred_element_type=F32) + bs1
    gate1 = jax.nn.sigmoid(
        jnp.dot(jax.nn.sigmoid(s1p), wsv1_ref[...],
                preferred_element_type=F32) + bsv1)
    wv1 = wv1_ref[...]
    v1 = [jnp.dot(vh[c], wv1, preferred_element_type=F32) * gate1
          for c in range(3)]
    s1 = jnp.maximum(s1p, 0.0)

    wh2 = wh2_ref[...]
    vh2 = [jnp.dot(v1[c], wh2, preferred_element_type=F32) for c in range(3)]
    vn2 = jnp.sqrt(vh2[0] * vh2[0] + vh2[1] * vh2[1] + vh2[2] * vh2[2] + 1e-8)
    s2p = jnp.dot(jnp.concatenate([s1, vn2], axis=1), ws2_ref[...],
                  preferred_element_type=F32) + bs2
    gate2 = jax.nn.sigmoid(
        jnp.dot(jax.nn.sigmoid(s2p), wsv2_ref[...],
                preferred_element_type=F32) + bsv2)
    wv2 = wv2_ref[...]
    v2 = [jnp.dot(vh2[c], wv2, preferred_element_type=F32) * gate2
          for c in range(3)]
    s2 = jnp.maximum(s2p, 0.0)

    wh3 = wh3_ref[...]
    vh3 = [jnp.dot(v2[c], wh3, preferred_element_type=F32) for c in range(3)]
    vn3 = jnp.sqrt(vh3[0] * vh3[0] + vh3[1] * vh3[1] + vh3[2] * vh3[2] + 1e-8)
    s3 = jnp.dot(jnp.concatenate([s2, vn3], axis=1), ws3_ref[...],
                 preferred_element_type=F32) + bs3
    gate3 = jax.nn.sigmoid(
        jnp.dot(s3, wsv3_ref[...], preferred_element_type=F32) + bsv3)
    wv3 = wv3_ref[...]
    v3 = [jnp.dot(vh3[c], wv3, preferred_element_type=F32) * gate3
          for c in range(3)]
    os_ref[...] = s + s3
    ov_ref[...] = vcm + jnp.concatenate([v3[0], v3[1], v3[2]], axis=1)


def _prep_upd_weights(chain):
    p1, p2, p3 = chain

    def padv(b):
        return jnp.concatenate([b, jnp.zeros((S - V,), F32)])

    bias = jnp.stack([p1["bs"], p2["bs"], p3["bs"],
                      padv(p1["bsv"]), padv(p2["bsv"]), padv(p3["bsv"])])
    return (p1["ws"], p1["wh"], p1["wv"], p1["wsv"],
            p2["ws"], p2["wh"], p2["wv"], p2["wsv"],
            p3["ws"], p3["wh"], p3["wv"], p3["wsv"], bias)


def _upd(s_w, vw_cm, part0, part1, wargs, tile=256):
    n = s_w.shape[0]
    in_specs = [
        pl.BlockSpec((tile, S), lambda i: (i, 0)),
        pl.BlockSpec((tile, 3 * V), lambda i: (i, 0)),
        pl.BlockSpec((tile, DMSG), lambda i: (i, 0)),
        pl.BlockSpec((tile, DMSG), lambda i: (i, 0)),
    ] + [_full_spec(w.shape) for w in wargs]
    return pl.pallas_call(
        _upd_body,
        grid=(n // tile,),
        in_specs=in_specs,
        out_specs=[pl.BlockSpec((tile, S), lambda i: (i, 0)),
                   pl.BlockSpec((tile, 3 * V), lambda i: (i, 0))],
        out_shape=[jax.ShapeDtypeStruct((n, S), F32),
                   jax.ShapeDtypeStruct((n, 3 * V), F32)],
    )(s_w, vw_cm, part0, part1, *wargs)


def _sc_gather(table, idx_flat):
    b = idx_flat.shape[0]
    d = table.shape[1]
    bpw = b // 32
    nch = bpw // 128
    idx3 = idx_flat.reshape(32, nch, 128)
    mesh = plsc.VectorSubcoreMesh(core_axis_name="c", subcore_axis_name="s")

    @functools.partial(
        pl.kernel, mesh=mesh,
        out_type=jax.ShapeDtypeStruct((b, d), F32),
        scratch_types=[pltpu.VMEM((128,), jnp.int32),
                       pltpu.VMEM((128, d), F32),
                       pltpu.SemaphoreType.DMA],
        compiler_params=pltpu.CompilerParams(use_tc_tiling_on_sc=False),
    )
    def kfn(table_hbm, idx_hbm, out_hbm, idx_v, rows_v, sem):
        cid = lax.axis_index("c")
        sid = lax.axis_index("s")
        wid = sid * 2 + cid
        base = wid * bpw

        def body(j, carry):
            pltpu.sync_copy(idx_hbm.at[wid, j], idx_v)
            pltpu.async_copy(table_hbm.at[idx_v], rows_v, sem).wait()
            pltpu.sync_copy(rows_v, out_hbm.at[pl.ds(base + j * 128, 128)])
            return carry

        lax.fori_loop(0, nch, body, 0)

    return kfn(table, idx3)


def _sc_scatter_add(msgs, idx3s, zeros_blk):
    d = msgs[0].shape[1]
    nchs = tuple(i3.shape[1] for i3 in idx3s)
    rows = NWAT // 16
    mesh = plsc.VectorSubcoreMesh(core_axis_name="c", subcore_axis_name="s")

    @functools.partial(
        pl.kernel, mesh=mesh,
        out_type=jax.ShapeDtypeStruct((2, NWAT, d), F32),
        scratch_types=[pltpu.VMEM_SHARED((NWAT, d), F32),
                       pltpu.VMEM((max(nchs), 64), jnp.int32),
                       pltpu.VMEM((64, d), F32)],
        compiler_params=pltpu.CompilerParams(use_tc_tiling_on_sc=False),
    )
    def kfn(m0, m1, m2, i0, i1, i2, z_hbm, out_hbm, shared, idx_v, buf_v):
        cid = lax.axis_index("c")
        sid = lax.axis_index("s")
        wid = sid * 2 + cid
        pltpu.sync_copy(z_hbm, shared.at[pl.ds(sid * rows, rows)])
        plsc.subcore_barrier()
        for m_hbm, i_hbm, nch in ((m0, i0, nchs[0]), (m1, i1, nchs[1]),
                                  (m2, i2, nchs[2])):
            pltpu.sync_copy(i_hbm.at[wid], idx_v.at[pl.ds(0, nch)])
            base = wid * nch * 64
            for j in range(nch):
                pltpu.sync_copy(m_hbm.at[pl.ds(base + j * 64, 64)], buf_v)
                pltpu.sync_copy(buf_v, shared.at[idx_v.at[j]], add=True)
        plsc.subcore_barrier()
        pltpu.sync_copy(shared.at[pl.ds(sid * rows, rows)],
                        out_hbm.at[cid, pl.ds(sid * rows, rows)])

    return kfn(*msgs, *idx3s, zeros_blk)


def kernel(s_p, v_p, s_m, v_m, s_w, v_w, pos_p, pos_w, pos_m, params):
    pad8 = lambda p: jnp.pad(p, ((0, 0), (0, 5)))
    pp8, pm8, pw8 = pad8(pos_p), pad8(pos_m), pad8(pos_w)
    pwt = jnp.transpose(pw8)
    pw16 = jnp.pad(pos_w, ((0, 0), (0, 13)))

    idx_pw = _knn(pp8, pwt, KPW, False)
    idx_mw = _knn(pm8, pwt, KPW, False)
    idx_ww = _knn(pw8, pwt, KWW, True)
    dst_pw = idx_pw.reshape(-1)
    dst_mw = idx_mw.reshape(-1)
    dst_ww = idx_ww.reshape(-1)

    pd_pw = _sc_gather(pw16, dst_pw)
    pd_mw = _sc_gather(pw16, dst_mw)
    pd_ww = _sc_gather(pw16, dst_ww)
    es_pw = _edge_static(pp8, pd_pw, KPW, 32)
    es_mw = _edge_static(pm8, pd_mw, KPW, 32)
    es_ww = _edge_static(pw8, pd_ww, KWW, 64)

    tocm = lambda v: jnp.transpose(v, (0, 2, 1)).reshape(v.shape[0], 3 * V)
    vp_cm, vm_cm, vw_cm = tocm(v_p), tocm(v_m), tocm(v_w)

    idx3_pw = dst_pw.reshape(32, -1, 64)
    idx3_mw = dst_mw.reshape(32, -1, 64)
    idx3_ww = dst_ww.reshape(32, -1, 64)
    zeros_blk = jnp.zeros((NWAT // 16, DMSG), F32)

    sw, vwc = s_w, vw_cm
    for bp in params["blocks"]:
        wsrc_pw, wdst_pw, args_pw = _prep_msg_weights(bp["msg"]["pw"])
        wsrc_mw, wdst_mw, args_mw = _prep_msg_weights(bp["msg"]["mw"])
        wsrc_ww, wdst_ww, args_ww = _prep_msg_weights(bp["msg"]["ww"])

        b_src_pw = _mm(s_p, wsrc_pw)
        b_src_mw = _mm(s_m, wsrc_mw)
        wat_cat = jnp.concatenate([wdst_pw, wdst_mw, wdst_ww, wsrc_ww],
                                  axis=1)
        wat = _mm(sw, wat_cat)
        bdst_pw, bdst_mw = wat[:, 0:S], wat[:, S:2 * S]
        bdst_ww, bsrc_ww = wat[:, 2 * S:3 * S], wat[:, 3 * S:4 * S]

        g_pw = _sc_gather(jnp.concatenate([bdst_pw, vwc], axis=1), dst_pw)
        g_mw = _sc_gather(jnp.concatenate([bdst_mw, vwc], axis=1), dst_mw)
        g_ww = _sc_gather(jnp.concatenate([bdst_ww, vwc], axis=1), dst_ww)

        m_pw = _msg(args_pw, b_src_pw, vp_cm, g_pw, es_pw, KPW, 32)
        m_mw = _msg(args_mw, b_src_mw, vm_cm, g_mw, es_mw, KPW, 32)
        m_ww = _msg(args_ww, bsrc_ww, vwc, g_ww, es_ww, KWW, 64)

        parts = _sc_scatter_add((m_pw, m_mw, m_ww),
                                (idx3_pw, idx3_mw, idx3_ww), zeros_blk)
        uargs = _prep_upd_weights(bp["upd"])
        sw, vwc = _upd(sw, vwc, parts[0], parts[1], uargs)

    v_out = vwc.reshape(NWAT, 3, V).transpose(0, 2, 1)
    return (sw, v_out)
